# baseline (speedup 1.0000x reference)
import functools

import jax
import jax.numpy as jnp
from jax import lax
from jax.experimental import pallas as pl
from jax.experimental.pallas import tpu as pltpu
from jax.experimental.pallas import tpu_sc as plsc

_N = 10000
_E = 320000
_D = 128
_NRAD = 16
_NMOL = 64
_INV_SQRT2 = 0.7071067811865475


def _silu(t):
    return t * jax.nn.sigmoid(t)


_BE = 3200
_BR = _BE // 128
_NB = _E // _BE


def _edge_body(ea_ref, rbf_ref, fin_ref, v_ref,
               wro_ref, wre_ref, wrf_ref,
               wfa0_ref, wfb0_ref, wfa1_ref, wfb1_ref, wof_ref,
               xm_ref, fx_ref, fy_ref, fz_ref):
    ea = ea_ref[...]
    rbf_out = jnp.dot(rbf_ref[...], wro_ref[...],
                      preferred_element_type=jnp.float32)
    xm_ref[...] = ea * jnp.dot(rbf_out, wre_ref[...],
                               preferred_element_type=jnp.float32)
    h = _silu(jnp.dot(ea, wfa0_ref[...], preferred_element_type=jnp.float32))
    h = _silu(jnp.dot(h, wfb0_ref[...], preferred_element_type=jnp.float32))
    xf = (ea + h) * _INV_SQRT2
    h = _silu(jnp.dot(xf, wfa1_ref[...], preferred_element_type=jnp.float32))
    h = _silu(jnp.dot(h, wfb1_ref[...], preferred_element_type=jnp.float32))
    xf = (xf + h) * _INV_SQRT2
    xf = xf * jnp.dot(rbf_out, wrf_ref[...], preferred_element_type=jnp.float32)
    f = jnp.dot(xf, wof_ref[...], preferred_element_type=jnp.float32)
    fl = f + fin_ref[...]
    fvt = jnp.transpose(fl * v_ref[...])
    fx_ref[...] = fvt[0:1, :].reshape((1, _BR, 128))
    fy_ref[...] = fvt[1:2, :].reshape((1, _BR, 128))
    fz_ref[...] = fvt[2:3, :].reshape((1, _BR, 128))


def _edge_compute(edge_attr, rbf, f_in, v, w_ro, w_re, w_rf,
                  wfa0, wfb0, wfa1, wfb1, w_of):
    eb = lambda i: (i, 0)
    e1 = lambda i: (i, 0, 0)
    wb = lambda i: (0, 0)
    return pl.pallas_call(
        _edge_body,
        grid=(_E // _BE,),
        in_specs=[
            pl.BlockSpec((_BE, _D), eb),
            pl.BlockSpec((_BE, _NRAD), eb),
            pl.BlockSpec((_BE, 1), eb),
            pl.BlockSpec((_BE, 3), eb),
            pl.BlockSpec((_NRAD, _NRAD), wb),
            pl.BlockSpec((_NRAD, _D), wb),
            pl.BlockSpec((_NRAD, _D), wb),
            pl.BlockSpec((_D, _D), wb),
            pl.BlockSpec((_D, _D), wb),
            pl.BlockSpec((_D, _D), wb),
            pl.BlockSpec((_D, _D), wb),
            pl.BlockSpec((_D, 1), wb),
        ],
        out_specs=[pl.BlockSpec((_BE, _D), eb),
                   pl.BlockSpec((1, _BR, 128), e1),
                   pl.BlockSpec((1, _BR, 128), e1),
                   pl.BlockSpec((1, _BR, 128), e1)],
        out_shape=[jax.ShapeDtypeStruct((_E, _D), jnp.float32),
                   jax.ShapeDtypeStruct((_NB, _BR, 128), jnp.float32),
                   jax.ShapeDtypeStruct((_NB, _BR, 128), jnp.float32),
                   jax.ShapeDtypeStruct((_NB, _BR, 128), jnp.float32)],
    )(edge_attr, rbf, f_in, v,
      w_ro, w_re, w_rf, wfa0, wfb0, wfa1, wfb1, w_of)


_NC = 2
_NS = 16
_NW = _NC * _NS
_CH = 256
_SB = 128
_NCHUNK = _E // _CH
_NP = 10240
_RPT = _NP // _NS


def _sc_body(dst_hbm, xm_hbm, fx_hbm, fy_hbm, fz_hbm, zx_hbm, z1_hbm,
             outx_hbm, ofx_hbm, ofy_hbm, ofz_hbm,
             ix0, ix1, rows0, rows1,
             fv0x, fv0y, fv0z, fv1x, fv1y, fv1z,
             sem0, sem1, xacc, fax, fay, faz):
    ixb = (ix0, ix1)
    rowsb = (rows0, rows1)
    fvb = ((fv0x, fv0y, fv0z), (fv1x, fv1y, fv1z))
    semb = (sem0, sem1)
    c = lax.axis_index("c")
    s = lax.axis_index("s")
    w = s * _NC + c
    r0 = s * _RPT

    def issue_loads(b, e):
        pltpu.async_copy(dst_hbm.at[pl.ds(e, _SB)], ixb[b], semb[b])
        pltpu.async_copy(xm_hbm.at[pl.ds(e, _SB)], rowsb[b], semb[b])
        pltpu.async_copy(fx_hbm.at[pl.ds(e, _SB)], fvb[b][0], semb[b])
        pltpu.async_copy(fy_hbm.at[pl.ds(e, _SB)], fvb[b][1], semb[b])
        pltpu.async_copy(fz_hbm.at[pl.ds(e, _SB)], fvb[b][2], semb[b])

    def wait_loads(b):
        pltpu.make_async_copy(dst_hbm.at[pl.ds(0, _SB)], ixb[b], semb[b]).wait()
        pltpu.make_async_copy(xm_hbm.at[pl.ds(0, _SB)], rowsb[b], semb[b]).wait()
        pltpu.make_async_copy(fx_hbm.at[pl.ds(0, _SB)], fvb[b][0], semb[b]).wait()
        pltpu.make_async_copy(fy_hbm.at[pl.ds(0, _SB)], fvb[b][1], semb[b]).wait()
        pltpu.make_async_copy(fz_hbm.at[pl.ds(0, _SB)], fvb[b][2], semb[b]).wait()

    def scatter(b):
        pltpu.sync_copy(rowsb[b], xacc.at[ixb[b]], add=True)
        pltpu.sync_copy(fvb[b][0], fax.at[ixb[b]], add=True)
        pltpu.sync_copy(fvb[b][1], fay.at[ixb[b]], add=True)
        pltpu.sync_copy(fvb[b][2], faz.at[ixb[b]], add=True)

    issue_loads(0, w * _CH)
    pltpu.sync_copy(zx_hbm.at[pl.ds(r0, _RPT)], xacc.at[pl.ds(r0, _RPT)])
    pltpu.sync_copy(z1_hbm.at[pl.ds(r0, _RPT)], fax.at[pl.ds(r0, _RPT)])
    pltpu.sync_copy(z1_hbm.at[pl.ds(r0, _RPT)], fay.at[pl.ds(r0, _RPT)])
    pltpu.sync_copy(z1_hbm.at[pl.ds(r0, _RPT)], faz.at[pl.ds(r0, _RPT)])
    plsc.subcore_barrier()

    my_count = (_NCHUNK + _NW - 1 - w) // _NW
    e_last = _E - _SB

    def chunk_body(t, carry):
        base = (t * _NW + w) * _CH
        wait_loads(0)
        issue_loads(1, base + _SB)
        scatter(0)
        wait_loads(1)
        e_next = jnp.minimum(((t + 1) * _NW + w) * _CH, e_last)
        issue_loads(0, e_next)
        scatter(1)
        return carry

    lax.fori_loop(0, my_count, chunk_body, 0)
    wait_loads(0)

    plsc.subcore_barrier()
    pltpu.sync_copy(xacc.at[pl.ds(r0, _RPT)],
                    outx_hbm.at[pl.ds(c * _NP + r0, _RPT)])
    pltpu.sync_copy(fax.at[pl.ds(r0, _RPT)],
                    ofx_hbm.at[pl.ds(c * _NP + r0, _RPT)])
    pltpu.sync_copy(fay.at[pl.ds(r0, _RPT)],
                    ofy_hbm.at[pl.ds(c * _NP + r0, _RPT)])
    pltpu.sync_copy(faz.at[pl.ds(r0, _RPT)],
                    ofz_hbm.at[pl.ds(c * _NP + r0, _RPT)])


_sc_scatter = functools.partial(
    pl.kernel,
    out_type=[jax.ShapeDtypeStruct((_NC * _NP, _D), jnp.float32),
              jax.ShapeDtypeStruct((_NC * _NP,), jnp.float32),
              jax.ShapeDtypeStruct((_NC * _NP,), jnp.float32),
              jax.ShapeDtypeStruct((_NC * _NP,), jnp.float32)],
    mesh=plsc.VectorSubcoreMesh(core_axis_name="c", subcore_axis_name="s"),
    scratch_types=(
        [pltpu.VMEM((_SB,), jnp.int32)] * 2
        + [pltpu.VMEM((_SB, _D), jnp.float32)] * 2
        + [pltpu.VMEM((_SB,), jnp.float32)] * 6
        + [pltpu.SemaphoreType.DMA] * 2
        + [pltpu.VMEM_SHARED((_NP, _D), jnp.float32)]
        + [pltpu.VMEM_SHARED((_NP,), jnp.float32)] * 3
    ),
)(_sc_body)


def _node_body(px_ref, ein_ref, bidx_ref, w1_ref, wa0_ref, wb0_ref,
               wa1_ref, wb1_ref, woe_ref, emol_ref):
    xe = px_ref[0:_N, :] + px_ref[_NP:_NP + _N, :]
    xe = _silu(jnp.dot(xe, w1_ref[...], preferred_element_type=jnp.float32))
    h = _silu(jnp.dot(xe, wa0_ref[...], preferred_element_type=jnp.float32))
    h = _silu(jnp.dot(h, wb0_ref[...], preferred_element_type=jnp.float32))
    xe = (xe + h) * _INV_SQRT2
    h = _silu(jnp.dot(xe, wa1_ref[...], preferred_element_type=jnp.float32))
    h = _silu(jnp.dot(h, wb1_ref[...], preferred_element_type=jnp.float32))
    xe = (xe + h) * _INV_SQRT2
    e_atom = jnp.dot(xe, woe_ref[...], preferred_element_type=jnp.float32)
    e_atom = e_atom + ein_ref[...]
    mol = lax.broadcasted_iota(jnp.int32, (_N, _NMOL), 1)
    mask = (bidx_ref[...] == mol).astype(jnp.float32)
    emol_ref[...] = lax.dot_general(
        mask, e_atom, (((0,), (0,)), ((), ())),
        preferred_element_type=jnp.float32)


def _node_compute(px, e_in, bidx, w1, wa0, wb0, wa1, wb1, woe):
    return pl.pallas_call(
        _node_body,
        out_shape=jax.ShapeDtypeStruct((_NMOL, 1), jnp.float32),
    )(px, e_in, bidx, w1, wa0, wb0, wa1, wb1, woe)


def kernel(x, edge_attr, edge_index, rbf, batch_idx, E_in, F_in, v, y,
           W_rbf_out, W_rbf_E, W1_E, WresE0a, WresE0b, WresE1a, WresE1b,
           W_out_E, WresF0a, WresF0b, WresF1a, WresF1b, W_rbf_F, W_out_F):
    xm, fx2, fy2, fz2 = _edge_compute(
        edge_attr, rbf, F_in, v,
        W_rbf_out, W_rbf_E, W_rbf_F, WresF0a, WresF0b, WresF1a,
        WresF1b, W_out_F)
    fx = fx2.reshape(_E)
    fy = fy2.reshape(_E)
    fz = fz2.reshape(_E)
    dst1d = edge_index[1]
    zx = jnp.zeros((_NP, _D), jnp.float32)
    z1 = jnp.zeros((_NP,), jnp.float32)
    px, ofx, ofy, ofz = _sc_scatter(dst1d, xm, fx, fy, fz, zx, z1)
    emol = _node_compute(px, E_in, batch_idx.reshape(_N, 1), W1_E,
                         WresE0a, WresE0b, WresE1a, WresE1b, W_out_E)
    fj = jnp.stack([ofx[0:_N] + ofx[_NP:_NP + _N],
                    ofy[0:_N] + ofy[_NP:_NP + _N],
                    ofz[0:_N] + ofz[_NP:_NP + _N]], axis=1)
    return (emol, fj, y)


# speedup vs baseline: 1.3296x; 1.3296x over previous
import functools

import jax
import jax.numpy as jnp
from jax import lax
from jax.experimental import pallas as pl
from jax.experimental.pallas import tpu as pltpu
from jax.experimental.pallas import tpu_sc as plsc

_N = 10000
_E = 320000
_D = 128
_NRAD = 16
_NMOL = 64
_INV_SQRT2 = 0.7071067811865475


def _silu(t):
    return t * jax.nn.sigmoid(t)


_BE = 3200
_BR = _BE // 128
_NB = _E // _BE


def _edge_body(ea_ref, rbf_ref, fin_ref, vx_ref, vy_ref, vz_ref,
               wro_ref, wre_ref, wrf_ref,
               wfa0_ref, wfb0_ref, wfa1_ref, wfb1_ref, wof_ref,
               xm_ref, fx_ref, fy_ref, fz_ref):
    ea = ea_ref[...]
    rbf_out = jnp.dot(rbf_ref[...], wro_ref[...],
                      preferred_element_type=jnp.float32)
    xm_ref[...] = ea * jnp.dot(rbf_out, wre_ref[...],
                               preferred_element_type=jnp.float32)
    h = _silu(jnp.dot(ea, wfa0_ref[...], preferred_element_type=jnp.float32))
    h = _silu(jnp.dot(h, wfb0_ref[...], preferred_element_type=jnp.float32))
    xf = (ea + h) * _INV_SQRT2
    h = _silu(jnp.dot(xf, wfa1_ref[...], preferred_element_type=jnp.float32))
    h = _silu(jnp.dot(h, wfb1_ref[...], preferred_element_type=jnp.float32))
    xf = (xf + h) * _INV_SQRT2
    xf = xf * jnp.dot(rbf_out, wrf_ref[...], preferred_element_type=jnp.float32)
    f = jnp.dot(xf, wof_ref[...], preferred_element_type=jnp.float32)
    fl = jnp.transpose(f).reshape((1, _BR, 128)) + fin_ref[...]
    fx_ref[...] = fl * vx_ref[...]
    fy_ref[...] = fl * vy_ref[...]
    fz_ref[...] = fl * vz_ref[...]


def _edge_compute(edge_attr, rbf, f_in1, vx, vy, vz, w_ro, w_re, w_rf,
                  wfa0, wfb0, wfa1, wfb1, w_of):
    eb = lambda i: (i, 0)
    e1 = lambda i: (i, 0, 0)
    wb = lambda i: (0, 0)
    return pl.pallas_call(
        _edge_body,
        grid=(_E // _BE,),
        in_specs=[
            pl.BlockSpec((_BE, _D), eb),
            pl.BlockSpec((_BE, _NRAD), eb),
            pl.BlockSpec((1, _BR, 128), e1),
            pl.BlockSpec((1, _BR, 128), e1),
            pl.BlockSpec((1, _BR, 128), e1),
            pl.BlockSpec((1, _BR, 128), e1),
            pl.BlockSpec((_NRAD, _NRAD), wb),
            pl.BlockSpec((_NRAD, _D), wb),
            pl.BlockSpec((_NRAD, _D), wb),
            pl.BlockSpec((_D, _D), wb),
            pl.BlockSpec((_D, _D), wb),
            pl.BlockSpec((_D, _D), wb),
            pl.BlockSpec((_D, _D), wb),
            pl.BlockSpec((_D, 1), wb),
        ],
        out_specs=[pl.BlockSpec((_BE, _D), eb),
                   pl.BlockSpec((1, _BR, 128), e1),
                   pl.BlockSpec((1, _BR, 128), e1),
                   pl.BlockSpec((1, _BR, 128), e1)],
        out_shape=[jax.ShapeDtypeStruct((_E, _D), jnp.float32),
                   jax.ShapeDtypeStruct((_NB, _BR, 128), jnp.float32),
                   jax.ShapeDtypeStruct((_NB, _BR, 128), jnp.float32),
                   jax.ShapeDtypeStruct((_NB, _BR, 128), jnp.float32)],
    )(edge_attr, rbf, f_in1, vx, vy, vz,
      w_ro, w_re, w_rf, wfa0, wfb0, wfa1, wfb1, w_of)


_NC = 2
_NS = 16
_NW = _NC * _NS
_CH = 256
_SB = 128
_NCHUNK = _E // _CH
_NP = 10240
_RPT = _NP // _NS


def _sc_body(dst_hbm, xm_hbm, fx_hbm, fy_hbm, fz_hbm, zx_hbm, z1_hbm,
             outx_hbm, ofx_hbm, ofy_hbm, ofz_hbm,
             ix0, ix1, rows0, rows1,
             fv0x, fv0y, fv0z, fv1x, fv1y, fv1z,
             sem0, sem1, xacc, fax, fay, faz):
    ixb = (ix0, ix1)
    rowsb = (rows0, rows1)
    fvb = ((fv0x, fv0y, fv0z), (fv1x, fv1y, fv1z))
    semb = (sem0, sem1)
    c = lax.axis_index("c")
    s = lax.axis_index("s")
    w = s * _NC + c
    r0 = s * _RPT

    def issue_loads(b, e):
        pltpu.async_copy(dst_hbm.at[pl.ds(e, _SB)], ixb[b], semb[b])
        pltpu.async_copy(xm_hbm.at[pl.ds(e, _SB)], rowsb[b], semb[b])
        pltpu.async_copy(fx_hbm.at[pl.ds(e, _SB)], fvb[b][0], semb[b])
        pltpu.async_copy(fy_hbm.at[pl.ds(e, _SB)], fvb[b][1], semb[b])
        pltpu.async_copy(fz_hbm.at[pl.ds(e, _SB)], fvb[b][2], semb[b])

    def wait_loads(b):
        pltpu.make_async_copy(dst_hbm.at[pl.ds(0, _SB)], ixb[b], semb[b]).wait()
        pltpu.make_async_copy(xm_hbm.at[pl.ds(0, _SB)], rowsb[b], semb[b]).wait()
        pltpu.make_async_copy(fx_hbm.at[pl.ds(0, _SB)], fvb[b][0], semb[b]).wait()
        pltpu.make_async_copy(fy_hbm.at[pl.ds(0, _SB)], fvb[b][1], semb[b]).wait()
        pltpu.make_async_copy(fz_hbm.at[pl.ds(0, _SB)], fvb[b][2], semb[b]).wait()

    def scatter(b):
        pltpu.sync_copy(rowsb[b], xacc.at[ixb[b]], add=True)
        pltpu.sync_copy(fvb[b][0], fax.at[ixb[b]], add=True)
        pltpu.sync_copy(fvb[b][1], fay.at[ixb[b]], add=True)
        pltpu.sync_copy(fvb[b][2], faz.at[ixb[b]], add=True)

    issue_loads(0, w * _CH)
    pltpu.sync_copy(zx_hbm.at[pl.ds(r0, _RPT)], xacc.at[pl.ds(r0, _RPT)])
    pltpu.sync_copy(z1_hbm.at[pl.ds(r0, _RPT)], fax.at[pl.ds(r0, _RPT)])
    pltpu.sync_copy(z1_hbm.at[pl.ds(r0, _RPT)], fay.at[pl.ds(r0, _RPT)])
    pltpu.sync_copy(z1_hbm.at[pl.ds(r0, _RPT)], faz.at[pl.ds(r0, _RPT)])
    plsc.subcore_barrier()

    my_count = (_NCHUNK + _NW - 1 - w) // _NW
    e_last = _E - _SB

    def chunk_body(t, carry):
        base = (t * _NW + w) * _CH
        wait_loads(0)
        issue_loads(1, base + _SB)
        scatter(0)
        wait_loads(1)
        e_next = jnp.minimum(((t + 1) * _NW + w) * _CH, e_last)
        issue_loads(0, e_next)
        scatter(1)
        return carry

    lax.fori_loop(0, my_count, chunk_body, 0)
    wait_loads(0)

    plsc.subcore_barrier()
    pltpu.sync_copy(xacc.at[pl.ds(r0, _RPT)],
                    outx_hbm.at[pl.ds(c * _NP + r0, _RPT)])
    pltpu.sync_copy(fax.at[pl.ds(r0, _RPT)],
                    ofx_hbm.at[pl.ds(c * _NP + r0, _RPT)])
    pltpu.sync_copy(fay.at[pl.ds(r0, _RPT)],
                    ofy_hbm.at[pl.ds(c * _NP + r0, _RPT)])
    pltpu.sync_copy(faz.at[pl.ds(r0, _RPT)],
                    ofz_hbm.at[pl.ds(c * _NP + r0, _RPT)])


_sc_scatter = functools.partial(
    pl.kernel,
    out_type=[jax.ShapeDtypeStruct((_NC * _NP, _D), jnp.float32),
              jax.ShapeDtypeStruct((_NC * _NP,), jnp.float32),
              jax.ShapeDtypeStruct((_NC * _NP,), jnp.float32),
              jax.ShapeDtypeStruct((_NC * _NP,), jnp.float32)],
    mesh=plsc.VectorSubcoreMesh(core_axis_name="c", subcore_axis_name="s"),
    scratch_types=(
        [pltpu.VMEM((_SB,), jnp.int32)] * 2
        + [pltpu.VMEM((_SB, _D), jnp.float32)] * 2
        + [pltpu.VMEM((_SB,), jnp.float32)] * 6
        + [pltpu.SemaphoreType.DMA] * 2
        + [pltpu.VMEM_SHARED((_NP, _D), jnp.float32)]
        + [pltpu.VMEM_SHARED((_NP,), jnp.float32)] * 3
    ),
)(_sc_body)


def _node_body(px_ref, ein_ref, bidx_ref, w1_ref, wa0_ref, wb0_ref,
               wa1_ref, wb1_ref, woe_ref, emol_ref):
    xe = px_ref[0:_N, :] + px_ref[_NP:_NP + _N, :]
    xe = _silu(jnp.dot(xe, w1_ref[...], preferred_element_type=jnp.float32))
    h = _silu(jnp.dot(xe, wa0_ref[...], preferred_element_type=jnp.float32))
    h = _silu(jnp.dot(h, wb0_ref[...], preferred_element_type=jnp.float32))
    xe = (xe + h) * _INV_SQRT2
    h = _silu(jnp.dot(xe, wa1_ref[...], preferred_element_type=jnp.float32))
    h = _silu(jnp.dot(h, wb1_ref[...], preferred_element_type=jnp.float32))
    xe = (xe + h) * _INV_SQRT2
    e_atom = jnp.dot(xe, woe_ref[...], preferred_element_type=jnp.float32)
    e_atom = e_atom + ein_ref[...]
    mol = lax.broadcasted_iota(jnp.int32, (_N, _NMOL), 1)
    mask = (bidx_ref[...] == mol).astype(jnp.float32)
    emol_ref[...] = lax.dot_general(
        mask, e_atom, (((0,), (0,)), ((), ())),
        preferred_element_type=jnp.float32)


def _node_compute(px, e_in, bidx, w1, wa0, wb0, wa1, wb1, woe):
    return pl.pallas_call(
        _node_body,
        out_shape=jax.ShapeDtypeStruct((_NMOL, 1), jnp.float32),
    )(px, e_in, bidx, w1, wa0, wb0, wa1, wb1, woe)


def kernel(x, edge_attr, edge_index, rbf, batch_idx, E_in, F_in, v, y,
           W_rbf_out, W_rbf_E, W1_E, WresE0a, WresE0b, WresE1a, WresE1b,
           W_out_E, WresF0a, WresF0b, WresF1a, WresF1b, W_rbf_F, W_out_F):
    vt = v.T
    fs = (_NB, _BR, 128)
    xm, fx2, fy2, fz2 = _edge_compute(
        edge_attr, rbf, F_in.reshape(fs),
        vt[0].reshape(fs), vt[1].reshape(fs), vt[2].reshape(fs),
        W_rbf_out, W_rbf_E, W_rbf_F, WresF0a, WresF0b, WresF1a,
        WresF1b, W_out_F)
    fx = fx2.reshape(_E)
    fy = fy2.reshape(_E)
    fz = fz2.reshape(_E)
    dst1d = edge_index[1]
    zx = jnp.zeros((_NP, _D), jnp.float32)
    z1 = jnp.zeros((_NP,), jnp.float32)
    px, ofx, ofy, ofz = _sc_scatter(dst1d, xm, fx, fy, fz, zx, z1)
    emol = _node_compute(px, E_in, batch_idx.reshape(_N, 1), W1_E,
                         WresE0a, WresE0b, WresE1a, WresE1b, W_out_E)
    fj = jnp.stack([ofx[0:_N] + ofx[_NP:_NP + _N],
                    ofy[0:_N] + ofy[_NP:_NP + _N],
                    ofz[0:_N] + ofz[_NP:_NP + _N]], axis=1)
    return (emol, fj, y)
